# baseline (speedup 1.0000x reference)
import functools

import jax
import jax.numpy as jnp
from jax import lax
from jax.experimental import pallas as pl
from jax.experimental.pallas import tpu as pltpu
from jax.experimental.pallas import tpu_sc as plsc

D = 128
C = 128
NSLOT = 6
NFLY = 4

_info = plsc.get_sparse_core_info()
_NC, _NS = _info.num_cores, _info.num_subcores
NW = _NC * _NS
NPAIR = NW // 2

_mesh = plsc.VectorSubcoreMesh(core_axis_name="c", subcore_axis_name="s")


def _make_gather(n_total: int):
    assert n_total % (NPAIR * C) == 0
    rpw = n_total // NPAIR
    nch = rpw // C

    @functools.partial(
        pl.kernel,
        mesh=_mesh,
        out_type=[
            jax.ShapeDtypeStruct((n_total, D), jnp.float32),
            jax.ShapeDtypeStruct((n_total, D), jnp.float32),
        ],
        scratch_types=[
            pltpu.VMEM((nch, C), jnp.int32),
            pltpu.VMEM((NSLOT, C, D), jnp.float32),
            pltpu.SemaphoreType.DMA,
            pltpu.SemaphoreType.DMA,
        ],
    )
    def gather_kernel(idx_hbm, cos_hbm, sin_hbm, cos_out, sin_out,
                      idx_v, buf, gsem, wsem):
        cid = lax.axis_index("c")
        sid = lax.axis_index("s")
        pair = cid * (_NS // 2) + sid // 2
        tsel = sid % 2
        base = pair * rpw
        pltpu.sync_copy(idx_hbm.at[pair], idx_v)

        def run_pipeline(tbl_hbm, out_hbm):
            def fire_gather(ch):
                return pltpu.async_copy(
                    tbl_hbm.at[idx_v.at[ch]], buf.at[ch % NSLOT], gsem)

            def fire_write(ch):
                return pltpu.async_copy(
                    buf.at[ch % NSLOT], out_hbm.at[pl.ds(base + ch * C, C)],
                    wsem)

            g = [None] * nch
            w = [None] * nch
            for ch in range(min(NFLY, nch)):
                g[ch] = fire_gather(ch)
            for ch in range(nch):
                nxt = ch + NFLY
                if nxt < nch:
                    prev = nxt - NSLOT
                    if prev >= 0:
                        w[prev].wait()
                    g[nxt] = fire_gather(nxt)
                g[ch].wait()
                w[ch] = fire_write(ch)
            for ch in range(max(0, nch - NSLOT), nch):
                if w[ch] is not None:
                    w[ch].wait()

        @pl.when(tsel == 0)
        def _():
            run_pipeline(cos_hbm, cos_out)

        @pl.when(tsel == 1)
        def _():
            run_pipeline(sin_hbm, sin_out)

    return gather_kernel


def kernel(position_ids, cos_emb, sin_emb):
    b, s = position_ids.shape
    n = b * s
    idx3 = position_ids.astype(jnp.int32).reshape(NPAIR, n // (NPAIR * C), C)
    g = _make_gather(n)
    cos_flat, sin_flat = g(idx3, cos_emb, sin_emb)
    return (cos_flat.reshape(b, s, D), sin_flat.reshape(b, s, D))


# speedup vs baseline: 1.0053x; 1.0053x over previous
import functools

import jax
import jax.numpy as jnp
from jax import lax
from jax.experimental import pallas as pl
from jax.experimental.pallas import tpu as pltpu
from jax.experimental.pallas import tpu_sc as plsc

D = 128
C = 128
NSLOT = 6
NFLY = 4

_info = plsc.get_sparse_core_info()
_NC, _NS = _info.num_cores, _info.num_subcores
NW = _NC * _NS
NPAIR = NW // 2

_mesh = plsc.VectorSubcoreMesh(core_axis_name="c", subcore_axis_name="s")


def _make_gather(b_len: int, s_len: int):
    n_total = b_len * s_len
    assert n_total % (NPAIR * C) == 0
    rpw = n_total // NPAIR
    nch = rpw // C
    assert s_len % rpw == 0

    @functools.partial(
        pl.kernel,
        mesh=_mesh,
        out_type=[
            jax.ShapeDtypeStruct((n_total, D), jnp.float32),
            jax.ShapeDtypeStruct((n_total, D), jnp.float32),
        ],
        scratch_types=[
            pltpu.VMEM((rpw,), jnp.int32),
            pltpu.VMEM((NSLOT, C, D), jnp.float32),
            pltpu.SemaphoreType.DMA,
            pltpu.SemaphoreType.DMA,
        ],
    )
    def gather_kernel(idx_hbm, cos_hbm, sin_hbm, cos_out, sin_out,
                      idx_v, buf, gsem, wsem):
        cid = lax.axis_index("c")
        sid = lax.axis_index("s")
        pair = cid * (_NS // 2) + sid // 2
        tsel = sid % 2
        base = pair * rpw
        ppr = s_len // rpw
        pltpu.sync_copy(
            idx_hbm.at[pair // ppr, pl.ds((pair % ppr) * rpw, rpw)], idx_v)

        def run_pipeline(tbl_hbm, out_hbm):
            def fire_gather(ch):
                return pltpu.async_copy(
                    tbl_hbm.at[idx_v.at[pl.ds(ch * C, C)]],
                    buf.at[ch % NSLOT], gsem)

            def fire_write(ch):
                return pltpu.async_copy(
                    buf.at[ch % NSLOT], out_hbm.at[pl.ds(base + ch * C, C)],
                    wsem)

            g = [None] * nch
            w = [None] * nch
            for ch in range(min(NFLY, nch)):
                g[ch] = fire_gather(ch)
            for ch in range(nch):
                nxt = ch + NFLY
                if nxt < nch:
                    prev = nxt - NSLOT
                    if prev >= 0:
                        w[prev].wait()
                    g[nxt] = fire_gather(nxt)
                g[ch].wait()
                w[ch] = fire_write(ch)
            for ch in range(max(0, nch - NSLOT), nch):
                if w[ch] is not None:
                    w[ch].wait()

        @pl.when(tsel == 0)
        def _():
            run_pipeline(cos_hbm, cos_out)

        @pl.when(tsel == 1)
        def _():
            run_pipeline(sin_hbm, sin_out)

    return gather_kernel


def kernel(position_ids, cos_emb, sin_emb):
    b, s = position_ids.shape
    g = _make_gather(b, s)
    cos_flat, sin_flat = g(position_ids.astype(jnp.int32), cos_emb, sin_emb)
    return (cos_flat.reshape(b, s, D), sin_flat.reshape(b, s, D))
